# Initial kernel scaffold; baseline (speedup 1.0000x reference)
import functools

import jax
import jax.numpy as jnp
from jax.experimental import pallas as pl
from jax.experimental.pallas import tpu as pltpu

SIMOTA_Q = 10
W_CLS, W_GEOM, W_IOU, W_DIST, W_THETA = 4.0, 5.0, 2.0, 1.0, 2.0
INVALID = -100000.0


def _assign_body(preds_t_ref, ptan_ref, gtxs_ref, gsc_ref, ys_ref,
                 assigned_ref, matched_ref, *, N, M, S, img_w, img_h):
    pt = preds_t_ref[0]
    l0 = pt[0:1]
    l1 = pt[1:2]
    sy = pt[2:3]
    sx = pt[3:4]
    th = pt[4:5]
    plen = pt[5:6]
    dx = pt[6:6 + S]
    tanp = ptan_ref[0]
    ys = ys_ref[...]

    pred_xs = sx + (sy - ys) * tanp + dx
    pmask = (ys >= (sy - plen)) & (ys <= sy)
    pred_xs = jnp.where(pmask, pred_xs, INVALID)
    px1 = pred_xs - 15.0
    px2 = pred_xs + 15.0
    pinv = pred_xs <= (INVALID + 1.0)

    gxs = gtxs_ref[0]
    gsc = gsc_ref[0]

    iou_rows = []
    for m in range(M):
        t = gxs[:, m:m + 1]
        tinv = t <= (INVALID + 1.0)
        ovr = jnp.maximum(jnp.minimum(px2, t + 15.0) - jnp.maximum(px1, t - 15.0), 0.0)
        uni = jnp.maximum(px2, t + 15.0) - jnp.minimum(px1, t - 15.0)
        inval = pinv | tinv
        ovr = jnp.where(inval, 0.0, ovr)
        uni = jnp.where(inval, 0.0, uni)
        iou = ovr.sum(axis=0, keepdims=True) / (uni.sum(axis=0, keepdims=True) + 1e-9)
        iou_rows.append(iou)
    ious = jnp.concatenate(iou_rows, axis=0)

    g_sy = gsc[:, 0:1]
    g_sx = gsc[:, 1:2]
    g_th = gsc[:, 2:3]
    g_mk = gsc[:, 3:4] > 0.0

    ious = jnp.where(g_mk, ious, 0.0)

    dxm = (sx - g_sx) * (1.0 / (img_w - 1))
    dym = (sy - g_sy) * (1.0 / (img_h - 1))
    dist = jnp.sqrt(dxm * dxm + dym * dym + 1e-8)
    thc = jnp.abs((th - g_th) * (1.0 / 90.0))
    mx = jnp.maximum(l0, l1)
    e0 = jnp.exp(l0 - mx)
    e1 = jnp.exp(l1 - mx)
    score = e1 / (e0 + e1)
    cls = -jnp.log(jnp.maximum(score, 1e-8))

    total = (W_CLS * cls
             + W_GEOM * (W_DIST * dist + W_THETA * thc)
             + W_IOU * (1.0 - ious))
    total = jnp.where(g_mk, total, 100000.0)

    idxN = jax.lax.broadcasted_iota(jnp.int32, (M, N), 1)

    vals = ious
    s10 = jnp.zeros((M, 1), jnp.float32)
    for _ in range(SIMOTA_Q):
        mv = jnp.max(vals, axis=1, keepdims=True)
        s10 = s10 + mv
        first = jnp.min(jnp.where(vals == mv, idxN, N), axis=1, keepdims=True)
        vals = jnp.where(idxN == first, -jnp.inf, vals)
    dyn_k = jnp.clip(s10.astype(jnp.int32), 1, N)

    cvals = total
    sel = jnp.zeros((M, N), jnp.bool_)
    for i in range(SIMOTA_Q):
        mv = jnp.min(cvals, axis=1, keepdims=True)
        first = jnp.min(jnp.where(cvals == mv, idxN, N), axis=1, keepdims=True)
        chosen = idxN == first
        sel = sel | (chosen & (i < dyn_k))
        cvals = jnp.where(chosen, jnp.inf, cvals)
    sel = sel & g_mk

    cnt = sel.astype(jnp.float32).sum(axis=0, keepdims=True)
    cost_sel = jnp.where(sel, total, jnp.inf)
    minc = jnp.min(cost_sel, axis=0, keepdims=True)
    mrow = jax.lax.broadcasted_iota(jnp.int32, (M, N), 0)
    min_m = jnp.min(jnp.where(cost_sel == minc, mrow, M), axis=0, keepdims=True)
    first_m = jnp.min(jnp.where(sel, mrow, M), axis=0, keepdims=True)
    assigned = cnt > 0.0
    matched = jnp.where(cnt > 1.0, min_m, first_m)
    matched = jnp.where(assigned, matched, -1)

    assigned_ref[0] = assigned.astype(jnp.int32)
    matched_ref[0] = matched


@functools.partial(jax.jit, static_argnums=(3, 4))
def kernel(preds, targets, masks, img_w, img_h):
    B, N, F = preds.shape
    M = targets.shape[1]
    S = F - 6

    sample_ys = jnp.linspace(img_h - 1, 0, S)
    ptan = jnp.clip(jnp.tan(jnp.deg2rad(preds[..., 4])), -1000.0, 1000.0)
    ptan = ptan[:, None, :]

    gt_start_y = targets[..., 2]
    gt_start_x = targets[..., 3]
    gt_theta = targets[..., 4]
    gt_delta_x = targets[..., 6:]
    gt_tan = jnp.clip(jnp.tan(jnp.deg2rad(gt_theta)), -1000.0, 1000.0)
    gt_xs = (gt_start_x[..., None]
             + (gt_start_y[..., None] - sample_ys.reshape(1, 1, -1)) * gt_tan[..., None]
             + gt_delta_x)
    ginv = (gt_delta_x < -10000.0) | (gt_xs < 0) | (gt_xs >= img_w)
    gt_xs = jnp.where(ginv, INVALID, gt_xs)
    gt_xs_t = jnp.transpose(gt_xs, (0, 2, 1))

    gsc = jnp.stack([gt_start_y, gt_start_x, gt_theta,
                     masks.astype(jnp.float32)] + [jnp.zeros_like(gt_theta)] * 4,
                    axis=-1)

    preds_t = jnp.transpose(preds, (0, 2, 1))
    ys_col = sample_ys.reshape(S, 1)

    body = functools.partial(_assign_body, N=N, M=M, S=S, img_w=img_w, img_h=img_h)
    out_shape = [jax.ShapeDtypeStruct((B, 1, N), jnp.int32),
                 jax.ShapeDtypeStruct((B, 1, N), jnp.int32)]
    assigned, matched = pl.pallas_call(
        body,
        grid=(B,),
        in_specs=[
            pl.BlockSpec((1, F, N), lambda b: (b, 0, 0)),
            pl.BlockSpec((1, 1, N), lambda b: (b, 0, 0)),
            pl.BlockSpec((1, S, M), lambda b: (b, 0, 0)),
            pl.BlockSpec((1, M, 8), lambda b: (b, 0, 0)),
            pl.BlockSpec((S, 1), lambda b: (0, 0)),
        ],
        out_specs=[
            pl.BlockSpec((1, 1, N), lambda b: (b, 0, 0)),
            pl.BlockSpec((1, 1, N), lambda b: (b, 0, 0)),
        ],
        out_shape=out_shape,
        compiler_params=pltpu.CompilerParams(
            dimension_semantics=("arbitrary",),
        ),
    )(preds_t, ptan, gt_xs_t, gsc, ys_col)

    return assigned.reshape(B, N).astype(jnp.bool_), matched.reshape(B, N)


# speedup vs baseline: 2.4533x; 2.4533x over previous
import functools

import jax
import jax.numpy as jnp
from jax.experimental import pallas as pl
from jax.experimental.pallas import tpu as pltpu

SIMOTA_Q = 10
W_CLS, W_GEOM, W_IOU, W_DIST, W_THETA = 4.0, 5.0, 2.0, 1.0, 2.0
INVALID = -100000.0


def _assign_body(preds_t_ref, ptan_ref, gtxs_ref, gsc_ref, ys_ref,
                 assigned_ref, matched_ref, *, N, M, S):
    pt = preds_t_ref[0]
    l0 = pt[0:1]
    l1 = pt[1:2]
    sy = pt[2:3]
    sx = pt[3:4]
    th = pt[4:5]
    plen = pt[5:6]
    dx = pt[6:6 + S]
    tanp = ptan_ref[0]
    ys = ys_ref[...]

    pred_xs = sx + (sy - ys) * tanp + dx
    pmask = (ys >= (sy - plen)) & (ys <= sy)
    pred_xs = jnp.where(pmask, pred_xs, INVALID)
    px1 = pred_xs - 15.0
    px2 = pred_xs + 15.0
    pinv = pred_xs <= (INVALID + 1.0)

    gxs = gtxs_ref[0]
    gsc = gsc_ref[0]

    iou_rows = []
    for m in range(M):
        t = gxs[:, m:m + 1]
        tinv = t <= (INVALID + 1.0)
        ovr = jnp.maximum(jnp.minimum(px2, t + 15.0) - jnp.maximum(px1, t - 15.0), 0.0)
        uni = jnp.maximum(px2, t + 15.0) - jnp.minimum(px1, t - 15.0)
        inval = pinv | tinv
        ovr = jnp.where(inval, 0.0, ovr)
        uni = jnp.where(inval, 0.0, uni)
        iou = ovr.sum(axis=0, keepdims=True) / (uni.sum(axis=0, keepdims=True) + 1e-9)
        iou_rows.append(iou)
    ious = jnp.concatenate(iou_rows, axis=0)

    g_sy = gsc[:, 0:1]
    g_sx = gsc[:, 1:2]
    g_th = gsc[:, 2:3]
    g_mk = gsc[:, 3:4] > 0.0
    inv_w = gsc[0:1, 4:5]
    inv_h = gsc[0:1, 5:6]

    ious = jnp.where(g_mk, ious, 0.0)

    dxm = (sx - g_sx) * inv_w
    dym = (sy - g_sy) * inv_h
    dist = jnp.sqrt(dxm * dxm + dym * dym + 1e-8)
    thc = jnp.abs((th - g_th) * (1.0 / 90.0))
    mx = jnp.maximum(l0, l1)
    e0 = jnp.exp(l0 - mx)
    e1 = jnp.exp(l1 - mx)
    score = e1 / (e0 + e1)
    cls = -jnp.log(jnp.maximum(score, 1e-8))

    total = (W_CLS * cls
             + W_GEOM * (W_DIST * dist + W_THETA * thc)
             + W_IOU * (1.0 - ious))
    total = jnp.where(g_mk, total, 100000.0)

    idxN = jax.lax.broadcasted_iota(jnp.int32, (M, N), 1)

    vals = ious
    s10 = jnp.zeros((M, 1), jnp.float32)
    for _ in range(SIMOTA_Q):
        mv = jnp.max(vals, axis=1, keepdims=True)
        s10 = s10 + mv
        first = jnp.min(jnp.where(vals == mv, idxN, N), axis=1, keepdims=True)
        vals = jnp.where(idxN == first, -jnp.inf, vals)
    dyn_k = jnp.clip(s10.astype(jnp.int32), 1, N)

    cvals = total
    sel = jnp.zeros((M, N), jnp.bool_)
    for i in range(SIMOTA_Q):
        mv = jnp.min(cvals, axis=1, keepdims=True)
        first = jnp.min(jnp.where(cvals == mv, idxN, N), axis=1, keepdims=True)
        chosen = idxN == first
        sel = sel | (chosen & (i < dyn_k))
        cvals = jnp.where(chosen, jnp.inf, cvals)
    sel = sel & g_mk

    cnt = sel.astype(jnp.float32).sum(axis=0, keepdims=True)
    cost_sel = jnp.where(sel, total, jnp.inf)
    minc = jnp.min(cost_sel, axis=0, keepdims=True)
    mrow = jax.lax.broadcasted_iota(jnp.int32, (M, N), 0)
    min_m = jnp.min(jnp.where(cost_sel == minc, mrow, M), axis=0, keepdims=True)
    first_m = jnp.min(jnp.where(sel, mrow, M), axis=0, keepdims=True)
    assigned = cnt > 0.0
    matched = jnp.where(cnt > 1.0, min_m, first_m)
    matched = jnp.where(assigned, matched, -1)

    assigned_ref[0] = assigned.astype(jnp.int32)
    matched_ref[0] = matched


def kernel(preds, targets, masks, img_w, img_h):
    B, N, F = preds.shape
    M = targets.shape[1]
    S = F - 6

    sample_ys = jnp.linspace(img_h - 1, 0, S)
    ptan = jnp.clip(jnp.tan(jnp.deg2rad(preds[..., 4])), -1000.0, 1000.0)
    ptan = ptan[:, None, :]

    gt_start_y = targets[..., 2]
    gt_start_x = targets[..., 3]
    gt_theta = targets[..., 4]
    gt_delta_x = targets[..., 6:]
    gt_tan = jnp.clip(jnp.tan(jnp.deg2rad(gt_theta)), -1000.0, 1000.0)
    gt_xs = (gt_start_x[..., None]
             + (gt_start_y[..., None] - sample_ys.reshape(1, 1, -1)) * gt_tan[..., None]
             + gt_delta_x)
    ginv = (gt_delta_x < -10000.0) | (gt_xs < 0) | (gt_xs >= img_w)
    gt_xs = jnp.where(ginv, INVALID, gt_xs)
    gt_xs_t = jnp.transpose(gt_xs, (0, 2, 1))

    ones = jnp.ones_like(gt_theta)
    gsc = jnp.stack([gt_start_y, gt_start_x, gt_theta,
                     masks.astype(jnp.float32),
                     ones * (1.0 / (img_w - 1)), ones * (1.0 / (img_h - 1)),
                     jnp.zeros_like(gt_theta), jnp.zeros_like(gt_theta)],
                    axis=-1)

    preds_t = jnp.transpose(preds, (0, 2, 1))
    ys_col = sample_ys.reshape(S, 1)

    body = functools.partial(_assign_body, N=N, M=M, S=S)
    out_shape = [jax.ShapeDtypeStruct((B, 1, N), jnp.int32),
                 jax.ShapeDtypeStruct((B, 1, N), jnp.int32)]
    assigned, matched = pl.pallas_call(
        body,
        grid=(B,),
        in_specs=[
            pl.BlockSpec((1, F, N), lambda b: (b, 0, 0)),
            pl.BlockSpec((1, 1, N), lambda b: (b, 0, 0)),
            pl.BlockSpec((1, S, M), lambda b: (b, 0, 0)),
            pl.BlockSpec((1, M, 8), lambda b: (b, 0, 0)),
            pl.BlockSpec((S, 1), lambda b: (0, 0)),
        ],
        out_specs=[
            pl.BlockSpec((1, 1, N), lambda b: (b, 0, 0)),
            pl.BlockSpec((1, 1, N), lambda b: (b, 0, 0)),
        ],
        out_shape=out_shape,
        compiler_params=pltpu.CompilerParams(
            dimension_semantics=("arbitrary",),
        ),
    )(preds_t, ptan, gt_xs_t, gsc, ys_col)

    return assigned.reshape(B, N).astype(jnp.bool_), matched.reshape(B, N)


# speedup vs baseline: 2.5589x; 1.0430x over previous
import functools

import jax
import jax.numpy as jnp
from jax.experimental import pallas as pl
from jax.experimental.pallas import tpu as pltpu

SIMOTA_Q = 10
W_CLS, W_GEOM, W_IOU, W_DIST, W_THETA = 4.0, 5.0, 2.0, 1.0, 2.0
INVALID = -100000.0


def _assign_body(preds_t_ref, ptan_ref, gtxs_ref, gsc_ref, ys_ref,
                 assigned_ref, matched_ref, *, N, M, S):
    pt = jnp.transpose(preds_t_ref[0])

    l0 = pt[0:1]
    l1 = pt[1:2]
    sy = pt[2:3]
    sx = pt[3:4]
    th = pt[4:5]
    plen = pt[5:6]
    dx = pt[6:6 + S]
    tanp = ptan_ref[0]
    ys = ys_ref[...]

    pred_xs = sx + (sy - ys) * tanp + dx
    pmask = (ys >= (sy - plen)) & (ys <= sy)
    pred_xs = jnp.where(pmask, pred_xs, INVALID)
    px1 = pred_xs - 15.0
    px2 = pred_xs + 15.0
    pinv = pred_xs <= (INVALID + 1.0)

    gxs = gtxs_ref[0]
    gsc = gsc_ref[0]

    iou_rows = []
    for m in range(M):
        t = gxs[:, m:m + 1]
        tinv = t <= (INVALID + 1.0)
        ovr = jnp.maximum(jnp.minimum(px2, t + 15.0) - jnp.maximum(px1, t - 15.0), 0.0)
        uni = jnp.maximum(px2, t + 15.0) - jnp.minimum(px1, t - 15.0)
        inval = pinv | tinv
        ovr = jnp.where(inval, 0.0, ovr)
        uni = jnp.where(inval, 0.0, uni)
        iou = ovr.sum(axis=0, keepdims=True) / (uni.sum(axis=0, keepdims=True) + 1e-9)
        iou_rows.append(iou)
    ious = jnp.concatenate(iou_rows, axis=0)

    g_sy = gsc[:, 0:1]
    g_sx = gsc[:, 1:2]
    g_th = gsc[:, 2:3]
    g_mk = gsc[:, 3:4] > 0.0
    inv_w = gsc[0:1, 4:5]
    inv_h = gsc[0:1, 5:6]

    ious = jnp.where(g_mk, ious, 0.0)

    dxm = (sx - g_sx) * inv_w
    dym = (sy - g_sy) * inv_h
    dist = jnp.sqrt(dxm * dxm + dym * dym + 1e-8)
    thc = jnp.abs((th - g_th) * (1.0 / 90.0))
    mx = jnp.maximum(l0, l1)
    e0 = jnp.exp(l0 - mx)
    e1 = jnp.exp(l1 - mx)
    score = e1 / (e0 + e1)
    cls = -jnp.log(jnp.maximum(score, 1e-8))

    total = (W_CLS * cls
             + W_GEOM * (W_DIST * dist + W_THETA * thc)
             + W_IOU * (1.0 - ious))
    total = jnp.where(g_mk, total, 100000.0)

    idxN = jax.lax.broadcasted_iota(jnp.int32, (M, N), 1)

    vals = ious
    s10 = jnp.zeros((M, 1), jnp.float32)
    for _ in range(SIMOTA_Q):
        mv = jnp.max(vals, axis=1, keepdims=True)
        s10 = s10 + mv
        first = jnp.min(jnp.where(vals == mv, idxN, N), axis=1, keepdims=True)
        vals = jnp.where(idxN == first, -jnp.inf, vals)
    dyn_k = jnp.clip(s10.astype(jnp.int32), 1, N)

    cvals = total
    sel = jnp.zeros((M, N), jnp.bool_)
    for i in range(SIMOTA_Q):
        mv = jnp.min(cvals, axis=1, keepdims=True)
        first = jnp.min(jnp.where(cvals == mv, idxN, N), axis=1, keepdims=True)
        chosen = idxN == first
        sel = sel | (chosen & (i < dyn_k))
        cvals = jnp.where(chosen, jnp.inf, cvals)
    sel = sel & g_mk

    cnt = sel.astype(jnp.float32).sum(axis=0, keepdims=True)
    cost_sel = jnp.where(sel, total, jnp.inf)
    minc = jnp.min(cost_sel, axis=0, keepdims=True)
    mrow = jax.lax.broadcasted_iota(jnp.int32, (M, N), 0)
    min_m = jnp.min(jnp.where(cost_sel == minc, mrow, M), axis=0, keepdims=True)
    first_m = jnp.min(jnp.where(sel, mrow, M), axis=0, keepdims=True)
    assigned = cnt > 0.0
    matched = jnp.where(cnt > 1.0, min_m, first_m)
    matched = jnp.where(assigned, matched, -1)

    assigned_ref[0] = assigned.astype(jnp.int32)
    matched_ref[0] = matched


def kernel(preds, targets, masks, img_w, img_h):
    B, N, F = preds.shape
    M = targets.shape[1]
    S = F - 6

    sample_ys = jnp.linspace(img_h - 1, 0, S)
    ptan = jnp.clip(jnp.tan(jnp.deg2rad(preds[..., 4])), -1000.0, 1000.0)
    ptan = ptan[:, None, :]

    gt_start_y = targets[..., 2]
    gt_start_x = targets[..., 3]
    gt_theta = targets[..., 4]
    gt_delta_x = targets[..., 6:]
    gt_tan = jnp.clip(jnp.tan(jnp.deg2rad(gt_theta)), -1000.0, 1000.0)
    gt_xs = (gt_start_x[..., None]
             + (gt_start_y[..., None] - sample_ys.reshape(1, 1, -1)) * gt_tan[..., None]
             + gt_delta_x)
    ginv = (gt_delta_x < -10000.0) | (gt_xs < 0) | (gt_xs >= img_w)
    gt_xs = jnp.where(ginv, INVALID, gt_xs)
    gt_xs_t = jnp.transpose(gt_xs, (0, 2, 1))

    ones = jnp.ones_like(gt_theta)
    gsc = jnp.stack([gt_start_y, gt_start_x, gt_theta,
                     masks.astype(jnp.float32),
                     ones * (1.0 / (img_w - 1)), ones * (1.0 / (img_h - 1)),
                     jnp.zeros_like(gt_theta), jnp.zeros_like(gt_theta)],
                    axis=-1)

    ys_col = sample_ys.reshape(S, 1)

    body = functools.partial(_assign_body, N=N, M=M, S=S)
    out_shape = [jax.ShapeDtypeStruct((B, 1, N), jnp.int32),
                 jax.ShapeDtypeStruct((B, 1, N), jnp.int32)]
    assigned, matched = pl.pallas_call(
        body,
        grid=(B,),
        in_specs=[
            pl.BlockSpec((1, N, F), lambda b: (b, 0, 0)),
            pl.BlockSpec((1, 1, N), lambda b: (b, 0, 0)),
            pl.BlockSpec((1, S, M), lambda b: (b, 0, 0)),
            pl.BlockSpec((1, M, 8), lambda b: (b, 0, 0)),
            pl.BlockSpec((S, 1), lambda b: (0, 0)),
        ],
        out_specs=[
            pl.BlockSpec((1, 1, N), lambda b: (b, 0, 0)),
            pl.BlockSpec((1, 1, N), lambda b: (b, 0, 0)),
        ],
        out_shape=out_shape,
        compiler_params=pltpu.CompilerParams(
            dimension_semantics=("arbitrary",),
        ),
    )(preds, ptan, gt_xs_t, gsc, ys_col)

    return assigned.reshape(B, N).astype(jnp.bool_), matched.reshape(B, N)


# speedup vs baseline: 2.6735x; 1.0448x over previous
import functools

import jax
import jax.numpy as jnp
from jax.experimental import pallas as pl
from jax.experimental.pallas import tpu as pltpu

SIMOTA_Q = 10
W_CLS, W_GEOM, W_IOU, W_DIST, W_THETA = 4.0, 5.0, 2.0, 1.0, 2.0
INVALID = -100000.0


def _assign_one(pt, tanp, gxs, gsc, ys, N, M, S):
    l0 = pt[0:1]
    l1 = pt[1:2]
    sy = pt[2:3]
    sx = pt[3:4]
    th = pt[4:5]
    plen = pt[5:6]
    dx = pt[6:6 + S]

    pred_xs = sx + (sy - ys) * tanp + dx
    pmask = (ys >= (sy - plen)) & (ys <= sy)
    pred_xs = jnp.where(pmask, pred_xs, INVALID)
    px1 = pred_xs - 15.0
    px2 = pred_xs + 15.0
    pinv = pred_xs <= (INVALID + 1.0)

    iou_rows = []
    for m in range(M):
        t = gxs[:, m:m + 1]
        tinv = t <= (INVALID + 1.0)
        ovr = jnp.maximum(jnp.minimum(px2, t + 15.0) - jnp.maximum(px1, t - 15.0), 0.0)
        uni = jnp.maximum(px2, t + 15.0) - jnp.minimum(px1, t - 15.0)
        inval = pinv | tinv
        ovr = jnp.where(inval, 0.0, ovr)
        uni = jnp.where(inval, 0.0, uni)
        iou = ovr.sum(axis=0, keepdims=True) / (uni.sum(axis=0, keepdims=True) + 1e-9)
        iou_rows.append(iou)
    ious = jnp.concatenate(iou_rows, axis=0)

    g_sy = gsc[:, 0:1]
    g_sx = gsc[:, 1:2]
    g_th = gsc[:, 2:3]
    g_mk = gsc[:, 3:4] > 0.0
    inv_w = gsc[0:1, 4:5]
    inv_h = gsc[0:1, 5:6]

    ious = jnp.where(g_mk, ious, 0.0)

    dxm = (sx - g_sx) * inv_w
    dym = (sy - g_sy) * inv_h
    dist = jnp.sqrt(dxm * dxm + dym * dym + 1e-8)
    thc = jnp.abs((th - g_th) * (1.0 / 90.0))
    mx = jnp.maximum(l0, l1)
    e0 = jnp.exp(l0 - mx)
    e1 = jnp.exp(l1 - mx)
    score = e1 / (e0 + e1)
    cls = -jnp.log(jnp.maximum(score, 1e-8))

    total = (W_CLS * cls
             + W_GEOM * (W_DIST * dist + W_THETA * thc)
             + W_IOU * (1.0 - ious))
    total = jnp.where(g_mk, total, 100000.0)

    idxN = jax.lax.broadcasted_iota(jnp.int32, (M, N), 1)

    vals = ious
    s10 = jnp.zeros((M, 1), jnp.float32)
    for _ in range(SIMOTA_Q):
        mv = jnp.max(vals, axis=1, keepdims=True)
        s10 = s10 + mv
        first = jnp.min(jnp.where(vals == mv, idxN, N), axis=1, keepdims=True)
        vals = jnp.where(idxN == first, -jnp.inf, vals)
    dyn_k = jnp.clip(s10.astype(jnp.int32), 1, N)

    cvals = total
    sel = jnp.zeros((M, N), jnp.bool_)
    for i in range(SIMOTA_Q):
        mv = jnp.min(cvals, axis=1, keepdims=True)
        first = jnp.min(jnp.where(cvals == mv, idxN, N), axis=1, keepdims=True)
        chosen = idxN == first
        sel = sel | (chosen & (i < dyn_k))
        cvals = jnp.where(chosen, jnp.inf, cvals)
    sel = sel & g_mk

    cnt = sel.astype(jnp.float32).sum(axis=0, keepdims=True)
    cost_sel = jnp.where(sel, total, jnp.inf)
    minc = jnp.min(cost_sel, axis=0, keepdims=True)
    mrow = jax.lax.broadcasted_iota(jnp.int32, (M, N), 0)
    min_m = jnp.min(jnp.where(cost_sel == minc, mrow, M), axis=0, keepdims=True)
    first_m = jnp.min(jnp.where(sel, mrow, M), axis=0, keepdims=True)
    assigned = cnt > 0.0
    matched = jnp.where(cnt > 1.0, min_m, first_m)
    matched = jnp.where(assigned, matched, -1)
    return assigned.astype(jnp.int32), matched


def _assign_body(preds_ref, ptan_ref, gtxs_ref, gsc_ref, ys_ref,
                 assigned_ref, matched_ref, *, N, M, S, BPG):
    ys = ys_ref[...]
    for bi in range(BPG):
        pt = jnp.transpose(preds_ref[bi])
        a, mt = _assign_one(pt, ptan_ref[bi], gtxs_ref[bi], gsc_ref[bi],
                            ys, N, M, S)
        assigned_ref[bi] = a
        matched_ref[bi] = mt


def kernel(preds, targets, masks, img_w, img_h):
    B, N, F = preds.shape
    M = targets.shape[1]
    S = F - 6

    sample_ys = jnp.linspace(img_h - 1, 0, S)
    ptan = jnp.clip(jnp.tan(jnp.deg2rad(preds[..., 4])), -1000.0, 1000.0)
    ptan = ptan[:, None, :]

    gt_start_y = targets[..., 2]
    gt_start_x = targets[..., 3]
    gt_theta = targets[..., 4]
    gt_delta_x = targets[..., 6:]
    gt_tan = jnp.clip(jnp.tan(jnp.deg2rad(gt_theta)), -1000.0, 1000.0)
    gt_xs = (gt_start_x[..., None]
             + (gt_start_y[..., None] - sample_ys.reshape(1, 1, -1)) * gt_tan[..., None]
             + gt_delta_x)
    ginv = (gt_delta_x < -10000.0) | (gt_xs < 0) | (gt_xs >= img_w)
    gt_xs = jnp.where(ginv, INVALID, gt_xs)
    gt_xs_t = jnp.transpose(gt_xs, (0, 2, 1))

    ones = jnp.ones_like(gt_theta)
    gsc = jnp.stack([gt_start_y, gt_start_x, gt_theta,
                     masks.astype(jnp.float32),
                     ones * (1.0 / (img_w - 1)), ones * (1.0 / (img_h - 1)),
                     jnp.zeros_like(gt_theta), jnp.zeros_like(gt_theta)],
                    axis=-1)

    ys_col = sample_ys.reshape(S, 1)

    BPG = 2
    body = functools.partial(_assign_body, N=N, M=M, S=S, BPG=BPG)
    out_shape = [jax.ShapeDtypeStruct((B, 1, N), jnp.int32),
                 jax.ShapeDtypeStruct((B, 1, N), jnp.int32)]
    assigned, matched = pl.pallas_call(
        body,
        grid=(B // BPG,),
        in_specs=[
            pl.BlockSpec((BPG, N, F), lambda b: (b, 0, 0)),
            pl.BlockSpec((BPG, 1, N), lambda b: (b, 0, 0)),
            pl.BlockSpec((BPG, S, M), lambda b: (b, 0, 0)),
            pl.BlockSpec((BPG, M, 8), lambda b: (b, 0, 0)),
            pl.BlockSpec((S, 1), lambda b: (0, 0)),
        ],
        out_specs=[
            pl.BlockSpec((BPG, 1, N), lambda b: (b, 0, 0)),
            pl.BlockSpec((BPG, 1, N), lambda b: (b, 0, 0)),
        ],
        out_shape=out_shape,
        compiler_params=pltpu.CompilerParams(
            dimension_semantics=("arbitrary",),
        ),
    )(preds, ptan, gt_xs_t, gsc, ys_col)

    return assigned.reshape(B, N).astype(jnp.bool_), matched.reshape(B, N)
